# baseline (speedup 1.0000x reference)
import functools

import jax
import jax.numpy as jnp
from jax import lax
from jax.experimental import pallas as pl
from jax.experimental.pallas import tpu as pltpu
from jax.experimental.pallas import tpu_sc as plsc

B = 16384
EMB = 32
ROWP = 128
GPAD = 16
BLK = 2048
GRID = B // BLK

FCOLS = 1664
FGRID = 61


def _fmt_body(in_ref, out_ref):
    x = in_ref[...]
    eye = (lax.broadcasted_iota(jnp.int32, (EMB, ROWP), 0)
           == lax.broadcasted_iota(jnp.int32, (EMB, ROWP), 1)
           ).astype(jnp.float32)
    out_ref[...] = lax.dot_general(
        x, eye, (((0,), (0,)), ((), ())), preferred_element_type=jnp.float32)


@functools.cache
def _build_fmt(interpret=False):
    return pl.pallas_call(
        _fmt_body,
        grid=(FGRID,),
        in_specs=[pl.BlockSpec((EMB, FCOLS), lambda i: (0, i))],
        out_specs=pl.BlockSpec((FCOLS, ROWP), lambda i: (i, 0)),
        out_shape=jax.ShapeDtypeStruct((FGRID * FCOLS, ROWP), jnp.float32),
        interpret=interpret,
    )


@functools.cache
def _build_gather():
    info = plsc.get_sparse_core_info()
    nc, ns = info.num_cores, info.num_subcores
    nw = nc * ns
    b_per_w = B // nw
    mesh = plsc.VectorSubcoreMesh(core_axis_name="c", subcore_axis_name="s")

    @functools.partial(
        pl.kernel,
        mesh=mesh,
        out_type=jax.ShapeDtypeStruct((B, ROWP), jnp.float32),
        scratch_types=[
            pltpu.VMEM((b_per_w,), jnp.int32),
            pltpu.VMEM((b_per_w, ROWP), jnp.float32),
            pltpu.SemaphoreType.DMA,
        ],
        compiler_params=pltpu.CompilerParams(use_tc_tiling_on_sc=False),
    )
    def gather(table_hbm, idx_hbm, out_hbm, idx_v, rows_v, sem):
        wid = lax.axis_index("s") * nc + lax.axis_index("c")
        base = wid * b_per_w
        pltpu.sync_copy(idx_hbm.at[pl.ds(base, b_per_w)], idx_v)
        pltpu.async_copy(table_hbm.at[idx_v], rows_v, sem).wait()
        pltpu.sync_copy(rows_v, out_hbm.at[pl.ds(base, b_per_w)])

    return gather


def _mlp_body(cat_ref, u_ref, gt_ref, w1_ref, b1_ref, w2_ref, b2_ref,
              w3_ref, b3_ref, out_ref):
    u = u_ref[:, :EMB]
    cat = cat_ref[0, 0, :]
    col = lax.broadcasted_iota(jnp.int32, (BLK, GPAD), 1)
    onehot = (col == cat[:, None]).astype(jnp.float32)
    g = jnp.dot(onehot, gt_ref[...], preferred_element_type=jnp.float32)
    x = jnp.concatenate([u, g], axis=1)
    h = jnp.maximum(
        jnp.dot(x, w1_ref[...], preferred_element_type=jnp.float32)
        + b1_ref[...], 0.0)
    h = jnp.maximum(
        jnp.dot(h, w2_ref[...], preferred_element_type=jnp.float32)
        + b2_ref[...], 0.0)
    out = (jnp.dot(h, w3_ref[...], preferred_element_type=jnp.float32)
           + b3_ref[...])
    out_ref[...] = out.T


@functools.cache
def _build_mlp(interpret=False):
    full = lambda *shape: pl.BlockSpec(shape, lambda i: (0,) * len(shape))
    return pl.pallas_call(
        _mlp_body,
        grid=(GRID,),
        in_specs=[
            pl.BlockSpec((1, 1, BLK), lambda i: (i, 0, 0)),
            pl.BlockSpec((BLK, ROWP), lambda i: (i, 0)),
            full(GPAD, EMB),
            full(2 * EMB, 128), full(1, 128),
            full(128, 64), full(1, 64),
            full(64, EMB), full(1, EMB),
        ],
        out_specs=pl.BlockSpec((EMB, BLK), lambda i: (0, i)),
        out_shape=jax.ShapeDtypeStruct((EMB, B), jnp.float32),
        interpret=interpret,
    )


def kernel(customer_id, category_by_Gender, user_table, gender_table,
           W1, b1, W2, b2, W3, b3):
    cid = customer_id.astype(jnp.int32)
    cat = category_by_Gender.astype(jnp.int32).reshape(GRID, 1, BLK)
    table_p = _build_fmt()(user_table.T)
    user_emb = _build_gather()(table_p, cid)
    gt_pad = jnp.pad(gender_table, ((0, GPAD - gender_table.shape[0]), (0, 0)))
    out_t = _build_mlp()(
        cat, user_emb, gt_pad,
        W1, b1.reshape(1, -1), W2, b2.reshape(1, -1), W3, b3.reshape(1, -1))
    return out_t.T


# speedup vs baseline: 1.0777x; 1.0777x over previous
import functools

import jax
import jax.numpy as jnp
from jax import lax
from jax.experimental import pallas as pl
from jax.experimental.pallas import tpu as pltpu
from jax.experimental.pallas import tpu_sc as plsc

B = 16384
EMB = 32
ROWP = 128
GPAD = 16
BLK = 2048
GRID = B // BLK

FP = 512
FGRID = 49
QS = FGRID * FP


def _fmt_body(x0_ref, x1_ref, x2_ref, x3_ref, out_ref):
    i = pl.program_id(0)
    acc = None
    for k, ref in enumerate((x0_ref, x1_ref, x2_ref, x3_ref)):
        x = ref[...]
        if k == 3:
            gcol = ((3 * FGRID + i) * FP
                    + lax.broadcasted_iota(jnp.int32, (EMB, FP), 1))
            x = jnp.where(gcol < 100001, x, 0.0)
        lane = lax.broadcasted_iota(jnp.int32, (EMB, ROWP), 1)
        row = lax.broadcasted_iota(jnp.int32, (EMB, ROWP), 0)
        eye_k = (lane == row + k * EMB).astype(jnp.float32)
        part = lax.dot_general(x, eye_k, (((0,), (0,)), ((), ())),
                               preferred_element_type=jnp.float32)
        acc = part if acc is None else acc + part
    out_ref[...] = acc


@functools.cache
def _build_fmt(interpret=False):
    def in_spec(k):
        return pl.BlockSpec((EMB, FP), lambda i, k=k: (0, k * FGRID + i))
    return pl.pallas_call(
        _fmt_body,
        grid=(FGRID,),
        in_specs=[in_spec(0), in_spec(1), in_spec(2), in_spec(3)],
        out_specs=pl.BlockSpec((FP, ROWP), lambda i: (i, 0)),
        out_shape=jax.ShapeDtypeStruct((QS, ROWP), jnp.float32),
        interpret=interpret,
    )


@functools.cache
def _build_gather():
    info = plsc.get_sparse_core_info()
    nc, ns = info.num_cores, info.num_subcores
    nw = nc * ns
    b_per_w = B // nw
    mesh = plsc.VectorSubcoreMesh(core_axis_name="c", subcore_axis_name="s")

    @functools.partial(
        pl.kernel,
        mesh=mesh,
        out_type=jax.ShapeDtypeStruct((B, ROWP), jnp.float32),
        scratch_types=[
            pltpu.VMEM((b_per_w,), jnp.int32),
            pltpu.VMEM((b_per_w, ROWP), jnp.float32),
            pltpu.SemaphoreType.DMA,
        ],
        compiler_params=pltpu.CompilerParams(use_tc_tiling_on_sc=False),
    )
    def gather(table_hbm, idx_hbm, out_hbm, idx_v, rows_v, sem):
        wid = lax.axis_index("s") * nc + lax.axis_index("c")
        base = wid * b_per_w
        pltpu.sync_copy(idx_hbm.at[pl.ds(base, b_per_w)], idx_v)
        pltpu.async_copy(table_hbm.at[idx_v], rows_v, sem).wait()
        pltpu.sync_copy(rows_v, out_hbm.at[pl.ds(base, b_per_w)])

    return gather


def _mlp_body(cid_ref, cat_ref, u_ref, gt_ref, w1_ref, b1_ref, w2_ref,
              b2_ref, w3_ref, b3_ref, out_ref):
    u4 = u_ref[...]
    cid = cid_ref[0, 0, :]
    k = ((cid >= QS).astype(jnp.int32)
         + (cid >= 2 * QS).astype(jnp.int32)
         + (cid >= 3 * QS).astype(jnp.int32))
    k2 = k[:, None]
    u01 = jnp.where(k2 == 0, u4[:, 0:EMB], u4[:, EMB:2 * EMB])
    u23 = jnp.where(k2 == 2, u4[:, 2 * EMB:3 * EMB], u4[:, 3 * EMB:])
    u = jnp.where(k2 < 2, u01, u23)
    cat = cat_ref[0, 0, :]
    col = lax.broadcasted_iota(jnp.int32, (BLK, GPAD), 1)
    onehot = (col == cat[:, None]).astype(jnp.float32)
    g = jnp.dot(onehot, gt_ref[...], preferred_element_type=jnp.float32)
    x = jnp.concatenate([u, g], axis=1)
    h = jnp.maximum(
        jnp.dot(x, w1_ref[...], preferred_element_type=jnp.float32)
        + b1_ref[...], 0.0)
    h = jnp.maximum(
        jnp.dot(h, w2_ref[...], preferred_element_type=jnp.float32)
        + b2_ref[...], 0.0)
    out = (jnp.dot(h, w3_ref[...], preferred_element_type=jnp.float32)
           + b3_ref[...])
    out_ref[...] = out.T


@functools.cache
def _build_mlp(interpret=False):
    full = lambda *shape: pl.BlockSpec(shape, lambda i: (0,) * len(shape))
    return pl.pallas_call(
        _mlp_body,
        grid=(GRID,),
        in_specs=[
            pl.BlockSpec((1, 1, BLK), lambda i: (i, 0, 0)),
            pl.BlockSpec((1, 1, BLK), lambda i: (i, 0, 0)),
            pl.BlockSpec((BLK, ROWP), lambda i: (i, 0)),
            full(GPAD, EMB),
            full(2 * EMB, 128), full(1, 128),
            full(128, 64), full(1, 64),
            full(64, EMB), full(1, EMB),
        ],
        out_specs=pl.BlockSpec((EMB, BLK), lambda i: (0, i)),
        out_shape=jax.ShapeDtypeStruct((EMB, B), jnp.float32),
        interpret=interpret,
    )


def kernel(customer_id, category_by_Gender, user_table, gender_table,
           W1, b1, W2, b2, W3, b3):
    cid = customer_id.astype(jnp.int32)
    cid3 = cid.reshape(GRID, 1, BLK)
    cat = category_by_Gender.astype(jnp.int32).reshape(GRID, 1, BLK)
    tt = user_table.T
    table_p = _build_fmt()(tt, tt, tt, tt)
    kq = ((cid >= QS).astype(jnp.int32) + (cid >= 2 * QS).astype(jnp.int32)
          + (cid >= 3 * QS).astype(jnp.int32))
    user_emb = _build_gather()(table_p, cid - kq * QS)
    gt_pad = jnp.pad(gender_table, ((0, GPAD - gender_table.shape[0]), (0, 0)))
    out_t = _build_mlp()(
        cid3, cat, user_emb, gt_pad,
        W1, b1.reshape(1, -1), W2, b2.reshape(1, -1), W3, b3.reshape(1, -1))
    return out_t.T
